# Initial kernel scaffold; baseline (speedup 1.0000x reference)
import jax
import jax.numpy as jnp
from jax.experimental import pallas as pl


def kernel(x, edge_index, edge_weight, p, W0, W_ih, W_hh, b_ih, b_hh, W_conv1, b_conv1, W_lin, b_lin):
    raise NotImplementedError("write your pallas kernel here")


# speedup vs baseline: 50.5836x; 50.5836x over previous
import functools

import jax
import jax.numpy as jnp
from jax import lax
from jax.experimental import pallas as pl
from jax.experimental.pallas import tpu as pltpu
from jax.experimental.pallas import tpu_sc as plsc

N_NODES = 10000
N_EDGES = 320000
D = 128
F = 4

NC = 2
NS = 16
NW = NC * NS
EPW = N_EDGES // NW
EPW_PAD = 10112
CH = 128
NCHUNK = EPW_PAD // CH
NP = 10240
NPT = NP // NS

_mesh = lambda: plsc.VectorSubcoreMesh(core_axis_name="c", subcore_axis_name="s")


def _zero_fill(ref, nwords):
    zv = jnp.zeros((16,), ref.dtype)

    def body(i, _):
        ref[pl.ds(i * 16, 16)] = zv
        return 0

    lax.fori_loop(0, nwords // 16, body, 0)


def _deg_body(col_hbm, ew_hbm, out_hbm, colv, ewv, onev, idxb, zbuf, accw, accc):
    cid = lax.axis_index("c")
    sid = lax.axis_index("s")
    wid = cid * NS + sid

    _zero_fill(zbuf, NPT)
    pltpu.sync_copy(zbuf, accw.at[pl.ds(sid * NPT, NPT)])
    pltpu.sync_copy(zbuf, accc.at[pl.ds(sid * NPT, NPT)])

    pltpu.sync_copy(col_hbm.at[pl.ds(wid * EPW, EPW)], colv.at[pl.ds(0, EPW)])
    pltpu.sync_copy(ew_hbm.at[pl.ds(wid * EPW, EPW)], ewv.at[pl.ds(0, EPW)])

    def fill(i, _):
        lane = i * 16 + lax.iota(jnp.int32, 16)
        valid = lane < EPW
        onev[pl.ds(i * 16, 16)] = jnp.where(valid, 1.0, 0.0)
        ewv[pl.ds(i * 16, 16)] = jnp.where(valid, ewv[pl.ds(i * 16, 16)], 0.0)
        colv[pl.ds(i * 16, 16)] = jnp.where(valid, colv[pl.ds(i * 16, 16)], 0)
        return 0

    lax.fori_loop(EPW // 16, EPW_PAD // 16, fill, 0)

    def fill_ones(i, _):
        onev[pl.ds(i * 16, 16)] = jnp.full((16,), 1.0, jnp.float32)
        return 0

    lax.fori_loop(0, EPW // 16, fill_ones, 0)

    plsc.subcore_barrier()

    def chunk(c, _):
        off = c * CH
        for g in range(CH // 16):
            idxb[pl.ds(g * 16, 16)] = colv[pl.ds(off + g * 16, 16)]
        pltpu.sync_copy(ewv.at[pl.ds(off, CH)], accw.at[idxb], add=True)
        pltpu.sync_copy(onev.at[pl.ds(off, CH)], accc.at[idxb], add=True)
        return 0

    lax.fori_loop(0, NCHUNK, chunk, 0)

    plsc.subcore_barrier()

    sl = pl.ds(sid * NPT, NPT)
    pltpu.sync_copy(accw.at[sl], out_hbm.at[cid, 0, sl])
    pltpu.sync_copy(accc.at[sl], out_hbm.at[cid, 1, sl])


_deg_kernel = functools.partial(
    pl.kernel,
    out_type=jax.ShapeDtypeStruct((NC, 2, NP), jnp.float32),
    mesh=_mesh(),
    compiler_params=pltpu.CompilerParams(needs_layout_passes=False),
    scratch_types=[
        pltpu.VMEM((EPW_PAD,), jnp.int32),
        pltpu.VMEM((EPW_PAD,), jnp.float32),
        pltpu.VMEM((EPW_PAD,), jnp.float32),
        pltpu.VMEM((CH,), jnp.int32),
        pltpu.VMEM((NPT,), jnp.float32),
        pltpu.VMEM_SHARED((NP,), jnp.float32),
        pltpu.VMEM_SHARED((NP,), jnp.float32),
    ],
)(_deg_body)


def _make_agg_body(weighted):
    def body(row_hbm, col_hbm, ew_hbm, feat_hbm, out_hbm,
             rowv, colv, ewv, featv, idxb, vb0, vb1, vb2, vb3, zbuf,
             acc0, acc1, acc2, acc3):
        cid = lax.axis_index("c")
        sid = lax.axis_index("s")
        wid = cid * NS + sid
        accs = (acc0, acc1, acc2, acc3)
        vbs = (vb0, vb1, vb2, vb3)

        _zero_fill(zbuf, NPT)
        for a in accs:
            pltpu.sync_copy(zbuf, a.at[pl.ds(sid * NPT, NPT)])

        pltpu.sync_copy(row_hbm.at[pl.ds(wid * EPW, EPW)], rowv.at[pl.ds(0, EPW)])
        pltpu.sync_copy(col_hbm.at[pl.ds(wid * EPW, EPW)], colv.at[pl.ds(0, EPW)])
        if weighted:
            pltpu.sync_copy(ew_hbm.at[pl.ds(wid * EPW, EPW)], ewv.at[pl.ds(0, EPW)])
        pltpu.sync_copy(feat_hbm, featv)

        def fill(i, _):
            lane = i * 16 + lax.iota(jnp.int32, 16)
            valid = lane < EPW
            rowv[pl.ds(i * 16, 16)] = jnp.where(valid, rowv[pl.ds(i * 16, 16)],
                                                N_NODES)
            colv[pl.ds(i * 16, 16)] = jnp.where(valid, colv[pl.ds(i * 16, 16)], 0)
            ewv[pl.ds(i * 16, 16)] = jnp.where(valid, ewv[pl.ds(i * 16, 16)], 0.0)
            return 0

        lax.fori_loop(EPW // 16, EPW_PAD // 16, fill, 0)

        plsc.subcore_barrier()

        def chunk(c, _):
            off = c * CH
            for g in range(CH // 16):
                sl = pl.ds(off + g * 16, 16)
                bl = pl.ds(g * 16, 16)
                r = rowv[sl]
                idxb[bl] = colv[sl]
                w = ewv[sl] if weighted else None
                r4 = r * 4
                for f in range(F):
                    vf = plsc.load_gather(featv, [r4 + f])
                    vbs[f][bl] = (w * vf) if weighted else vf
            for f in range(F):
                pltpu.sync_copy(vbs[f], accs[f].at[idxb], add=True)
            return 0

        lax.fori_loop(0, NCHUNK, chunk, 0)

        plsc.subcore_barrier()

        sl = pl.ds(sid * NPT, NPT)
        for f in range(F):
            pltpu.sync_copy(accs[f].at[sl], out_hbm.at[cid, f, sl])

    return body


def _make_agg_kernel(weighted):
    return functools.partial(
        pl.kernel,
        out_type=jax.ShapeDtypeStruct((NC, F, NP), jnp.float32),
        mesh=_mesh(),
        compiler_params=pltpu.CompilerParams(needs_layout_passes=False),
        scratch_types=[
            pltpu.VMEM((EPW_PAD,), jnp.int32),
            pltpu.VMEM((EPW_PAD,), jnp.int32),
            pltpu.VMEM((EPW_PAD,), jnp.float32),
            pltpu.VMEM((NP * F,), jnp.float32),
            pltpu.VMEM((CH,), jnp.int32),
            pltpu.VMEM((CH,), jnp.float32),
            pltpu.VMEM((CH,), jnp.float32),
            pltpu.VMEM((CH,), jnp.float32),
            pltpu.VMEM((CH,), jnp.float32),
            pltpu.VMEM((NPT,), jnp.float32),
            pltpu.VMEM_SHARED((NP,), jnp.float32),
            pltpu.VMEM_SHARED((NP,), jnp.float32),
            pltpu.VMEM_SHARED((NP,), jnp.float32),
            pltpu.VMEM_SHARED((NP,), jnp.float32),
        ],
    )(_make_agg_body(weighted))


_agg_w = _make_agg_kernel(True)
_agg_u = _make_agg_kernel(False)


_HI = lax.Precision.HIGHEST


def _sigmoid(v):
    return 1.0 / (1.0 + jnp.exp(-v))


def _gru_body(xp_ref, vals_ref, W0_ref, Wih_ref, Whh_ref, bih_ref, bhh_ref,
              Wc1_ref, o_ref):
    xt = xp_ref[...] * jnp.tanh(vals_ref[...])
    W0 = W0_ref[...]
    gi = lax.dot_general(xt, Wih_ref[...], (((1,), (1,)), ((), ())),
                         precision=_HI) + bih_ref[...]
    gh = lax.dot_general(W0, Whh_ref[...], (((1,), (1,)), ((), ())),
                         precision=_HI) + bhh_ref[...]
    r = _sigmoid(gi[:, 0:D] + gh[:, 0:D])
    z = _sigmoid(gi[:, D:2 * D] + gh[:, D:2 * D])
    n = jnp.tanh(gi[:, 2 * D:] + r * gh[:, 2 * D:])
    W = (1.0 - z) * n + z * W0
    o_ref[...] = jnp.dot(W, Wc1_ref[...], precision=_HI)


def _z_body(x_ref, wc_ref, d1_ref, o_ref):
    z = jnp.dot(x_ref[...], wc_ref[...], precision=_HI)
    o_ref[0:N_NODES, :] = z * d1_ref[...]
    o_ref[N_NODES:NP, :] = jnp.zeros((NP - N_NODES, F), jnp.float32)


def _mid_body(acc_ref, zs_ref, d1_ref, d2_ref, o_ref):
    o_ref[...] = d1_ref[...] * d2_ref[...] * (acc_ref[...] + 2.0 * zs_ref[...])


def _fin_body(acc_ref, ys_ref, d2_ref, bc_ref, wl_ref, bl_ref, hlin_ref, out_ref):
    h2 = d2_ref[...] * (acc_ref[...] + ys_ref[...]) + bc_ref[...]
    out_ref[...] = jnp.tanh(h2)
    hlin_ref[...] = jnp.dot(h2, wl_ref[...], precision=_HI) + bl_ref[...]


def kernel(x, edge_index, edge_weight, p, W0, W_ih, W_hh, b_ih, b_hh,
           W_conv1, b_conv1, W_lin, b_lin):
    row = edge_index[0]
    col = edge_index[1]

    degp = _deg_kernel(col, edge_weight)
    deg1 = degp[0, 0] + degp[1, 0] + 2.0
    deg2 = degp[0, 1] + degp[1, 1] + 1.0
    dinv1 = lax.rsqrt(deg1)[:, None]
    dinv2 = lax.rsqrt(deg2)[:, None]

    score = (x @ p) / jnp.linalg.norm(p)
    vals, perm = lax.top_k(score, D)
    Wc = pl.pallas_call(
        _gru_body,
        out_shape=jax.ShapeDtypeStruct((D, F), jnp.float32),
    )(x[perm], vals[:, None], W0, W_ih, W_hh, b_ih[None, :], b_hh[None, :],
      W_conv1)

    zs = pl.pallas_call(
        _z_body,
        out_shape=jax.ShapeDtypeStruct((NP, F), jnp.float32),
    )(x, Wc, dinv1[:N_NODES])

    part1 = _agg_w(row, col, edge_weight, zs.reshape(-1))
    acc1 = (part1[0] + part1[1]).T

    ys = pl.pallas_call(
        _mid_body,
        out_shape=jax.ShapeDtypeStruct((NP, F), jnp.float32),
    )(acc1, zs, dinv1, dinv2)

    part2 = _agg_u(row, col, edge_weight, ys.reshape(-1))
    acc2 = (part2[0] + part2[1]).T

    hlin, out = pl.pallas_call(
        _fin_body,
        out_shape=(
            jax.ShapeDtypeStruct((N_NODES, 1), jnp.float32),
            jax.ShapeDtypeStruct((N_NODES, F), jnp.float32),
        ),
    )(acc2[:N_NODES], ys[:N_NODES], dinv2[:N_NODES], b_conv1[None, :],
      W_lin, b_lin[None, :])
    return (hlin, out)


# speedup vs baseline: 63.3042x; 1.2515x over previous
import functools

import jax
import jax.numpy as jnp
from jax import lax
from jax.experimental import pallas as pl
from jax.experimental.pallas import tpu as pltpu
from jax.experimental.pallas import tpu_sc as plsc

N_NODES = 10000
N_EDGES = 320000
D = 128
F = 4

NC = 2
NS = 16
NW = NC * NS
EPW = N_EDGES // NW
EPW_PAD = 10240
CH = 128
NCHUNK = EPW_PAD // CH
NPAIR = NCHUNK // 2
NP = 10240
NPT = NP // NS

_mesh = lambda: plsc.VectorSubcoreMesh(core_axis_name="c", subcore_axis_name="s")


def _zero_fill(ref, nwords):
    zv = jnp.zeros((16,), ref.dtype)

    def body(i, _):
        ref[pl.ds(i * 16, 16)] = zv
        return 0

    lax.fori_loop(0, nwords // 16, body, 0)


def _deg_body(col_hbm, ew_hbm, out_hbm, colv, ewv, onev, idxb0, idxb1, zbuf,
              accw, accc, sem0, sem1):
    cid = lax.axis_index("c")
    sid = lax.axis_index("s")
    wid = cid * NS + sid

    _zero_fill(zbuf, NPT)
    pltpu.sync_copy(zbuf, accw.at[pl.ds(sid * NPT, NPT)])
    pltpu.sync_copy(zbuf, accc.at[pl.ds(sid * NPT, NPT)])

    pltpu.sync_copy(col_hbm.at[pl.ds(wid * EPW, EPW)], colv.at[pl.ds(0, EPW)])
    pltpu.sync_copy(ew_hbm.at[pl.ds(wid * EPW, EPW)], ewv.at[pl.ds(0, EPW)])

    def fill(i, _):
        lane = i * 16 + lax.iota(jnp.int32, 16)
        valid = lane < EPW
        onev[pl.ds(i * 16, 16)] = jnp.where(valid, 1.0, 0.0)
        ewv[pl.ds(i * 16, 16)] = jnp.where(valid, ewv[pl.ds(i * 16, 16)], 0.0)
        colv[pl.ds(i * 16, 16)] = jnp.where(valid, colv[pl.ds(i * 16, 16)], 0)
        return 0

    lax.fori_loop(EPW // 16, EPW_PAD // 16, fill, 0)

    def fill_ones(i, _):
        onev[pl.ds(i * 16, 16)] = jnp.full((16,), 1.0, jnp.float32)
        return 0

    lax.fori_loop(0, EPW // 16, fill_ones, 0)

    plsc.subcore_barrier()

    idxbs = (idxb0, idxb1)
    sems = (sem0, sem1)

    def chunkpair(c2, _):
        for b in range(2):
            off = (c2 * 2 + b) * CH

            @pl.when(c2 > 0)
            def _():
                pltpu.make_async_copy(ewv.at[pl.ds(0, CH)],
                                      accw.at[idxbs[b]], sems[b]).wait()
                pltpu.make_async_copy(onev.at[pl.ds(0, CH)],
                                      accc.at[idxbs[b]], sems[b]).wait()

            for g in range(CH // 16):
                idxbs[b][pl.ds(g * 16, 16)] = colv[pl.ds(off + g * 16, 16)]
            pltpu.async_copy(ewv.at[pl.ds(off, CH)], accw.at[idxbs[b]],
                             sems[b], add=True)
            pltpu.async_copy(onev.at[pl.ds(off, CH)], accc.at[idxbs[b]],
                             sems[b], add=True)
        return 0

    lax.fori_loop(0, NPAIR, chunkpair, 0)
    for b in range(2):
        pltpu.make_async_copy(ewv.at[pl.ds(0, CH)], accw.at[idxbs[b]],
                              sems[b]).wait()
        pltpu.make_async_copy(onev.at[pl.ds(0, CH)], accc.at[idxbs[b]],
                              sems[b]).wait()

    plsc.subcore_barrier()

    sl = pl.ds(sid * NPT, NPT)
    pltpu.sync_copy(accw.at[sl], out_hbm.at[cid, 0, sl])
    pltpu.sync_copy(accc.at[sl], out_hbm.at[cid, 1, sl])


_deg_kernel = functools.partial(
    pl.kernel,
    out_type=jax.ShapeDtypeStruct((NC, 2, NP), jnp.float32),
    mesh=_mesh(),
    compiler_params=pltpu.CompilerParams(needs_layout_passes=False),
    scratch_types=[
        pltpu.VMEM((EPW_PAD,), jnp.int32),
        pltpu.VMEM((EPW_PAD,), jnp.float32),
        pltpu.VMEM((EPW_PAD,), jnp.float32),
        pltpu.VMEM((CH,), jnp.int32),
        pltpu.VMEM((CH,), jnp.int32),
        pltpu.VMEM((NPT,), jnp.float32),
        pltpu.VMEM_SHARED((NP,), jnp.float32),
        pltpu.VMEM_SHARED((NP,), jnp.float32),
        pltpu.SemaphoreType.DMA,
        pltpu.SemaphoreType.DMA,
    ],
)(_deg_body)


def _make_agg_body(weighted):
    def body(row_hbm, col_hbm, ew_hbm, feat_hbm, out_hbm,
             rowv, colv, ewv, featv, idxb0, idxb1,
             va0, va1, va2, va3, vb0, vb1, vb2, vb3, zbuf,
             acc0, acc1, acc2, acc3, sem0, sem1):
        cid = lax.axis_index("c")
        sid = lax.axis_index("s")
        wid = cid * NS + sid
        accs = (acc0, acc1, acc2, acc3)
        vbs = ((va0, va1, va2, va3), (vb0, vb1, vb2, vb3))
        idxbs = (idxb0, idxb1)
        sems = (sem0, sem1)

        _zero_fill(zbuf, NPT)
        for a in accs:
            pltpu.sync_copy(zbuf, a.at[pl.ds(sid * NPT, NPT)])

        pltpu.sync_copy(row_hbm.at[pl.ds(wid * EPW, EPW)], rowv.at[pl.ds(0, EPW)])
        pltpu.sync_copy(col_hbm.at[pl.ds(wid * EPW, EPW)], colv.at[pl.ds(0, EPW)])
        if weighted:
            pltpu.sync_copy(ew_hbm.at[pl.ds(wid * EPW, EPW)], ewv.at[pl.ds(0, EPW)])
        pltpu.sync_copy(feat_hbm, featv)

        def fill(i, _):
            lane = i * 16 + lax.iota(jnp.int32, 16)
            valid = lane < EPW
            rowv[pl.ds(i * 16, 16)] = jnp.where(valid, rowv[pl.ds(i * 16, 16)],
                                                N_NODES)
            colv[pl.ds(i * 16, 16)] = jnp.where(valid, colv[pl.ds(i * 16, 16)], 0)
            ewv[pl.ds(i * 16, 16)] = jnp.where(valid, ewv[pl.ds(i * 16, 16)], 0.0)
            return 0

        lax.fori_loop(EPW // 16, EPW_PAD // 16, fill, 0)

        plsc.subcore_barrier()

        def chunkpair(c2, _):
            for b in range(2):
                off = (c2 * 2 + b) * CH

                @pl.when(c2 > 0)
                def _():
                    for f in range(F):
                        pltpu.make_async_copy(vbs[b][f], accs[f].at[idxbs[b]],
                                              sems[b]).wait()

                for g in range(CH // 16):
                    sl = pl.ds(off + g * 16, 16)
                    bl = pl.ds(g * 16, 16)
                    r = rowv[sl]
                    idxbs[b][bl] = colv[sl]
                    w = ewv[sl] if weighted else None
                    r4 = r * 4
                    for f in range(F):
                        vf = plsc.load_gather(featv, [r4 + f])
                        vbs[b][f][bl] = (w * vf) if weighted else vf
                for f in range(F):
                    pltpu.async_copy(vbs[b][f], accs[f].at[idxbs[b]],
                                     sems[b], add=True)
            return 0

        lax.fori_loop(0, NPAIR, chunkpair, 0)
        for b in range(2):
            for f in range(F):
                pltpu.make_async_copy(vbs[b][f], accs[f].at[idxbs[b]],
                                      sems[b]).wait()

        plsc.subcore_barrier()

        sl = pl.ds(sid * NPT, NPT)
        for f in range(F):
            pltpu.sync_copy(accs[f].at[sl], out_hbm.at[cid, f, sl])

    return body


def _make_agg_kernel(weighted):
    return functools.partial(
        pl.kernel,
        out_type=jax.ShapeDtypeStruct((NC, F, NP), jnp.float32),
        mesh=_mesh(),
        compiler_params=pltpu.CompilerParams(needs_layout_passes=False),
        scratch_types=[
            pltpu.VMEM((EPW_PAD,), jnp.int32),
            pltpu.VMEM((EPW_PAD,), jnp.int32),
            pltpu.VMEM((EPW_PAD,), jnp.float32),
            pltpu.VMEM((NP * F,), jnp.float32),
            pltpu.VMEM((CH,), jnp.int32),
            pltpu.VMEM((CH,), jnp.int32),
            pltpu.VMEM((CH,), jnp.float32),
            pltpu.VMEM((CH,), jnp.float32),
            pltpu.VMEM((CH,), jnp.float32),
            pltpu.VMEM((CH,), jnp.float32),
            pltpu.VMEM((CH,), jnp.float32),
            pltpu.VMEM((CH,), jnp.float32),
            pltpu.VMEM((CH,), jnp.float32),
            pltpu.VMEM((CH,), jnp.float32),
            pltpu.VMEM((NPT,), jnp.float32),
            pltpu.VMEM_SHARED((NP,), jnp.float32),
            pltpu.VMEM_SHARED((NP,), jnp.float32),
            pltpu.VMEM_SHARED((NP,), jnp.float32),
            pltpu.VMEM_SHARED((NP,), jnp.float32),
            pltpu.SemaphoreType.DMA,
            pltpu.SemaphoreType.DMA,
        ],
    )(_make_agg_body(weighted))


_agg_w = _make_agg_kernel(True)
_agg_u = _make_agg_kernel(False)


_HI = lax.Precision.HIGHEST


def _sigmoid(v):
    return 1.0 / (1.0 + jnp.exp(-v))


def _gru_body(xp_ref, vals_ref, W0_ref, Wih_ref, Whh_ref, bih_ref, bhh_ref,
              Wc1_ref, o_ref):
    xt = xp_ref[...] * jnp.tanh(vals_ref[...])
    W0 = W0_ref[...]
    gi = lax.dot_general(xt, Wih_ref[...], (((1,), (1,)), ((), ())),
                         precision=_HI) + bih_ref[...]
    gh = lax.dot_general(W0, Whh_ref[...], (((1,), (1,)), ((), ())),
                         precision=_HI) + bhh_ref[...]
    r = _sigmoid(gi[:, 0:D] + gh[:, 0:D])
    z = _sigmoid(gi[:, D:2 * D] + gh[:, D:2 * D])
    n = jnp.tanh(gi[:, 2 * D:] + r * gh[:, 2 * D:])
    W = (1.0 - z) * n + z * W0
    o_ref[...] = jnp.dot(W, Wc1_ref[...], precision=_HI)


def _z_body(x_ref, wc_ref, d1_ref, o_ref):
    z = jnp.dot(x_ref[...], wc_ref[...], precision=_HI)
    o_ref[0:N_NODES, :] = z * d1_ref[...]
    o_ref[N_NODES:NP, :] = jnp.zeros((NP - N_NODES, F), jnp.float32)


def _mid_body(acc_ref, zs_ref, d1_ref, d2_ref, o_ref):
    o_ref[...] = d1_ref[...] * d2_ref[...] * (acc_ref[...] + 2.0 * zs_ref[...])


def _fin_body(acc_ref, ys_ref, d2_ref, bc_ref, wl_ref, bl_ref, hlin_ref, out_ref):
    h2 = d2_ref[...] * (acc_ref[...] + ys_ref[...]) + bc_ref[...]
    out_ref[...] = jnp.tanh(h2)
    hlin_ref[...] = jnp.dot(h2, wl_ref[...], precision=_HI) + bl_ref[...]


def kernel(x, edge_index, edge_weight, p, W0, W_ih, W_hh, b_ih, b_hh,
           W_conv1, b_conv1, W_lin, b_lin):
    row = edge_index[0]
    col = edge_index[1]

    degp = _deg_kernel(col, edge_weight)
    deg1 = degp[0, 0] + degp[1, 0] + 2.0
    deg2 = degp[0, 1] + degp[1, 1] + 1.0
    dinv1 = lax.rsqrt(deg1)[:, None]
    dinv2 = lax.rsqrt(deg2)[:, None]

    score = (x @ p) / jnp.linalg.norm(p)
    vals, perm = lax.top_k(score, D)
    Wc = pl.pallas_call(
        _gru_body,
        out_shape=jax.ShapeDtypeStruct((D, F), jnp.float32),
    )(x[perm], vals[:, None], W0, W_ih, W_hh, b_ih[None, :], b_hh[None, :],
      W_conv1)

    zs = pl.pallas_call(
        _z_body,
        out_shape=jax.ShapeDtypeStruct((NP, F), jnp.float32),
    )(x, Wc, dinv1[:N_NODES])

    part1 = _agg_w(row, col, edge_weight, zs.reshape(-1))
    acc1 = (part1[0] + part1[1]).T

    ys = pl.pallas_call(
        _mid_body,
        out_shape=jax.ShapeDtypeStruct((NP, F), jnp.float32),
    )(acc1, zs, dinv1, dinv2)

    part2 = _agg_u(row, col, edge_weight, ys.reshape(-1))
    acc2 = (part2[0] + part2[1]).T

    hlin, out = pl.pallas_call(
        _fin_body,
        out_shape=(
            jax.ShapeDtypeStruct((N_NODES, 1), jnp.float32),
            jax.ShapeDtypeStruct((N_NODES, F), jnp.float32),
        ),
    )(acc2[:N_NODES], ys[:N_NODES], dinv2[:N_NODES], b_conv1[None, :],
      W_lin, b_lin[None, :])
    return (hlin, out)
